# baseline (speedup 1.0000x reference)
import functools

import jax
import jax.numpy as jnp
import numpy as np
from jax.experimental import pallas as pl
from jax.experimental.pallas import tpu as pltpu

_MAX_CAV = 5
_RECORD_LEN = np.array([5, 3, 4, 2, 5, 4, 3, 2], dtype=np.int32)
_C, _H, _W = 256, 48, 176
_B = len(_RECORD_LEN)
_CP = _C + 3
_NSLOT = _B * _MAX_CAV

_cum = np.concatenate([[0], np.cumsum(_RECORD_LEN)])
_src_list = []
_valid_list = []
for _b in range(_B):
    for _l in range(_MAX_CAV):
        if _l < _RECORD_LEN[_b]:
            _src_list.append(_cum[_b] + _l)
            _valid_list.append(1)
        else:
            _src_list.append(_cum[_b] + _RECORD_LEN[_b] - 1)
            _valid_list.append(0)
_SRC = np.array(_src_list, dtype=np.int32)
_VALID = np.array(_valid_list, dtype=np.int32)

_MASK = jnp.asarray(_VALID.reshape(_B, _MAX_CAV), dtype=jnp.int32)

_HT = 48
_NJ = _H // _HT


def _regroup_kernel(src_ref, valid_ref, x_ref, pe_ref, out_ref, scr_ref, sem):
    s = pl.program_id(1)
    del src_ref
    valid = valid_ref[s]

    @pl.when(valid == 1)
    def _store_feat():
        scr_ref[...] = jnp.transpose(x_ref[0], (0, 2, 1))
        for h in range(_HT):
            pltpu.make_async_copy(
                scr_ref.at[h], out_ref.at[0, 0, pl.ds(0, _C), h, :], sem
            ).start()
        for h in range(_HT):
            pltpu.make_async_copy(
                scr_ref.at[h], out_ref.at[0, 0, pl.ds(0, _C), h, :], sem
            ).wait()

    @pl.when(valid == 0)
    def _store_zero():
        out_ref[0, 0, 0:_C, :, :] = jnp.zeros((_C, _HT, _W), dtype=out_ref.dtype)

    pe_row = pe_ref[s, :]
    out_ref[0, 0, _C:_CP, :, :] = jnp.broadcast_to(
        pe_row[:, None, None], (3, _HT, _W)
    )


@functools.partial(jax.jit)
def kernel(spatial_features_2d, prior_encoding):
    xt = jnp.transpose(spatial_features_2d, (0, 2, 3, 1))
    pe2d = prior_encoding.reshape(_NSLOT, 3)

    grid_spec = pltpu.PrefetchScalarGridSpec(
        num_scalar_prefetch=2,
        grid=(_NJ, _NSLOT),
        in_specs=[
            pl.BlockSpec((1, _HT, _W, _C), lambda j, s, src, valid: (src[s], j, 0, 0)),
            pl.BlockSpec((_NSLOT, 3), lambda j, s, src, valid: (0, 0)),
        ],
        out_specs=pl.BlockSpec(
            (1, 1, _CP, _HT, _W),
            lambda j, s, src, valid: (s // _MAX_CAV, s % _MAX_CAV, 0, j, 0),
        ),
        scratch_shapes=[
            pltpu.VMEM((_HT, _C, _W), jnp.float32),
            pltpu.SemaphoreType.DMA,
        ],
    )

    out5 = pl.pallas_call(
        _regroup_kernel,
        grid_spec=grid_spec,
        out_shape=jax.ShapeDtypeStruct((_B, _MAX_CAV, _CP, _H, _W), jnp.float32),
        compiler_params=pltpu.CompilerParams(
            dimension_semantics=("arbitrary", "arbitrary"),
        ),
    )(jnp.asarray(_SRC), jnp.asarray(_VALID), xt, pe2d)

    regroup_feature = jnp.transpose(out5, (0, 1, 3, 4, 2))
    return regroup_feature, _MASK


# speedup vs baseline: 1.0156x; 1.0156x over previous
import functools

import jax
import jax.numpy as jnp
import numpy as np
from jax.experimental import pallas as pl
from jax.experimental.pallas import tpu as pltpu

_MAX_CAV = 5
_RECORD_LEN = np.array([5, 3, 4, 2, 5, 4, 3, 2], dtype=np.int32)
_C, _H, _W = 256, 48, 176
_B = len(_RECORD_LEN)
_CP = _C + 3
_NSLOT = _B * _MAX_CAV

_cum = np.concatenate([[0], np.cumsum(_RECORD_LEN)])
_src_list = []
_valid_list = []
for _b in range(_B):
    for _l in range(_MAX_CAV):
        if _l < _RECORD_LEN[_b]:
            _src_list.append(_cum[_b] + _l)
            _valid_list.append(1)
        else:
            _src_list.append(_cum[_b] + _RECORD_LEN[_b] - 1)
            _valid_list.append(0)
_SRC = np.array(_src_list, dtype=np.int32)
_VALID = np.array(_valid_list, dtype=np.int32)

_MASK = jnp.asarray(_VALID.reshape(_B, _MAX_CAV), dtype=jnp.int32)

_HT = 24
_NJ = _H // _HT


def _regroup_kernel(src_ref, valid_ref, x_ref, pe_ref, out_ref, scr_ref, sem):
    s = pl.program_id(1)
    del src_ref
    valid = valid_ref[s]

    @pl.when(valid == 1)
    def _store_feat():
        scr_ref[...] = jnp.transpose(x_ref[0], (0, 2, 1))
        for h in range(_HT):
            pltpu.make_async_copy(
                scr_ref.at[h], out_ref.at[0, 0, pl.ds(0, _C), h, :], sem
            ).start()
        for h in range(_HT):
            pltpu.make_async_copy(
                scr_ref.at[h], out_ref.at[0, 0, pl.ds(0, _C), h, :], sem
            ).wait()

    @pl.when(valid == 0)
    def _store_zero():
        out_ref[0, 0, 0:_C, :, :] = jnp.zeros((_C, _HT, _W), dtype=out_ref.dtype)

    pe_row = pe_ref[s, :]
    out_ref[0, 0, _C:_CP, :, :] = jnp.broadcast_to(
        pe_row[:, None, None], (3, _HT, _W)
    )


@functools.partial(jax.jit)
def kernel(spatial_features_2d, prior_encoding):
    xt = jnp.transpose(spatial_features_2d, (0, 2, 3, 1))
    pe2d = prior_encoding.reshape(_NSLOT, 3)

    grid_spec = pltpu.PrefetchScalarGridSpec(
        num_scalar_prefetch=2,
        grid=(_NJ, _NSLOT),
        in_specs=[
            pl.BlockSpec((1, _HT, _W, _C), lambda j, s, src, valid: (src[s], j, 0, 0)),
            pl.BlockSpec((_NSLOT, 3), lambda j, s, src, valid: (0, 0)),
        ],
        out_specs=pl.BlockSpec(
            (1, 1, _CP, _HT, _W),
            lambda j, s, src, valid: (s // _MAX_CAV, s % _MAX_CAV, 0, j, 0),
        ),
        scratch_shapes=[
            pltpu.VMEM((_HT, _C, _W), jnp.float32),
            pltpu.SemaphoreType.DMA,
        ],
    )

    out5 = pl.pallas_call(
        _regroup_kernel,
        grid_spec=grid_spec,
        out_shape=jax.ShapeDtypeStruct((_B, _MAX_CAV, _CP, _H, _W), jnp.float32),
        compiler_params=pltpu.CompilerParams(
            dimension_semantics=("arbitrary", "arbitrary"),
        ),
    )(jnp.asarray(_SRC), jnp.asarray(_VALID), xt, pe2d)

    regroup_feature = jnp.transpose(out5, (0, 1, 3, 4, 2))
    return regroup_feature, _MASK
